# baseline (speedup 1.0000x reference)
import functools

import jax
import jax.numpy as jnp
from jax import lax
from jax.experimental import pallas as pl
from jax.experimental.pallas import tpu as pltpu
from jax.experimental.pallas import tpu_sc as plsc

N = 10000
E = 320000
D = 128
NEG_SLOPE = 0.2

NW = 32
NPT = 320
NPAD = NW * NPT
EDGES = E + N
W = 2048
GC = 16
NBLK = W // GC

_ROWS = 1000


def _mm2_body(x_ref, wl_ref, bl_ref, wr_ref, br_ref, xl_ref, xr_ref):
    x = x_ref[...]
    xl_ref[...] = jnp.dot(x, wl_ref[...], preferred_element_type=jnp.float32) + bl_ref[...]
    xr_ref[...] = jnp.dot(x, wr_ref[...], preferred_element_type=jnp.float32) + br_ref[...]


@jax.jit
def _mm2(x, Wl, bl, Wr, br):
    grid = (N // _ROWS,)
    return pl.pallas_call(
        _mm2_body,
        grid=grid,
        in_specs=[
            pl.BlockSpec((_ROWS, D), lambda i: (i, 0)),
            pl.BlockSpec((D, D), lambda i: (0, 0)),
            pl.BlockSpec((D,), lambda i: (0,)),
            pl.BlockSpec((D, D), lambda i: (0, 0)),
            pl.BlockSpec((D,), lambda i: (0,)),
        ],
        out_specs=[
            pl.BlockSpec((_ROWS, D), lambda i: (i, 0)),
            pl.BlockSpec((_ROWS, D), lambda i: (i, 0)),
        ],
        out_shape=[
            jax.ShapeDtypeStruct((N, D), jnp.float32),
            jax.ShapeDtypeStruct((N, D), jnp.float32),
        ],
    )(x, Wl, bl, Wr, br)


def _post_body(attn_ref, bias_ref, g_ref, b_ref, out_ref):
    o = attn_ref[...] + bias_ref[...]
    mu = jnp.mean(o, axis=-1, keepdims=True)
    var = jnp.mean((o - mu) ** 2, axis=-1, keepdims=True)
    o = (o - mu) / jnp.sqrt(var + 1e-5) * g_ref[...] + b_ref[...]
    out_ref[...] = jnp.maximum(o, 0.0)


@jax.jit
def _post(attn, bias, g, b):
    grid = (N // _ROWS,)
    return pl.pallas_call(
        _post_body,
        grid=grid,
        in_specs=[
            pl.BlockSpec((_ROWS, D), lambda i: (i, 0)),
            pl.BlockSpec((D,), lambda i: (0,)),
            pl.BlockSpec((D,), lambda i: (0,)),
            pl.BlockSpec((D,), lambda i: (0,)),
        ],
        out_specs=pl.BlockSpec((_ROWS, D), lambda i: (i, 0)),
        out_shape=jax.ShapeDtypeStruct((N, D), jnp.float32),
    )(attn, bias, g, b)


def _lanesum(v):
    lanes = lax.iota(jnp.int32, 16)
    for stride in (1, 2, 4, 8):
        v = v + jnp.take_along_axis(v, lanes ^ stride, axis=0)
    return v


def _edge_body(xl_hbm, xr_hbm, src_hbm, dst_hbm, rp_hbm, att_hbm, out_hbm,
               att_v, xr_v, srcw_v, dstw_v, rows0_v, rows1_v, rp_v, out_v,
               sem0, sem1):
    c = lax.axis_index("c")
    s = lax.axis_index("s")
    wid = s * 2 + c
    base = wid * NPT

    pltpu.sync_copy(att_hbm, att_v)
    pltpu.sync_copy(xr_hbm.at[pl.ds(base * D, NPT * D)], xr_v)
    pltpu.sync_copy(rp_hbm.at[pl.ds(base, 8)], rp_v.at[pl.ds(0, 8)])
    pltpu.sync_copy(rp_hbm.at[pl.ds(base + NPT, 8)], rp_v.at[pl.ds(8, 8)])
    rpv = rp_v[...]
    e0 = rpv[0]
    e1 = rpv[8]
    e0a = (e0 // 8) * 8
    att_regs = [att_v[pl.ds(16 * j, 16)] for j in range(8)]

    nwnd = (e1 - e0a + (W - 1)) // W

    bufs = ((rows0_v, sem0), (rows1_v, sem1))

    def _issue(b, buf, sem):
        start = jnp.minimum(b * GC, W - GC)
        pltpu.async_copy(xl_hbm.at[srcw_v[pl.ds(start, GC)]], buf, sem)

    def _wait(buf, sem):
        pltpu.make_async_copy(xl_hbm.at[pl.ds(0, GC), :], buf, sem).wait()

    def window_body(w, carry):
        wstart = e0a + w * W
        pltpu.sync_copy(src_hbm.at[pl.ds(wstart, W)], srcw_v)
        pltpu.sync_copy(dst_hbm.at[pl.ds(wstart, W)], dstw_v)
        nb = jnp.minimum(NBLK, (e1 - wstart + (GC - 1)) // GC)
        _issue(0, *bufs[0])

        def pair_body(p, carry):
            for h in (0, 1):
                b = 2 * p + h
                rows_v, sem = bufs[h]
                nxt_v, nxt_sem = bufs[1 - h]
                _issue(b + 1, nxt_v, nxt_sem)
                _wait(rows_v, sem)
                dvec = dstw_v[pl.ds(jnp.minimum(b * GC, W - GC), GC)]
                eid0 = wstart + b * GC
                cur_n, m, d, accs = carry
                for l in range(GC):
                    e_id = eid0 + l
                    active = jnp.logical_and(e_id >= e0, e_id < e1)
                    d_e = dvec[l]
                    nl = jnp.clip(d_e - base, 0, NPT - 1)
                    row = [rows_v[l, pl.ds(16 * j, 16)] for j in range(8)]
                    part = None
                    for j in range(8):
                        z = row[j] + xr_v[pl.ds(nl * D + 16 * j, 16)]
                        lz = jnp.maximum(z, NEG_SLOPE * z)
                        t = lz * att_regs[j]
                        part = t if part is None else part + t
                    lvec = _lanesum(part)
                    is_new = d_e != cur_n
                    do_fin = jnp.logical_and(jnp.logical_and(is_new, active),
                                             cur_n >= 0)
                    cl = jnp.clip(cur_n - base, 0, NPT - 1)

                    @pl.when(do_fin)
                    def _():
                        inv = 1.0 / d
                        for j in range(8):
                            out_v[pl.ds(cl * D + 16 * j, 16)] = accs[j] * inv

                    m_new = jnp.where(is_new, lvec, jnp.maximum(m, lvec))
                    s_old = jnp.where(is_new, jnp.zeros((16,), jnp.float32),
                                      jnp.exp(m - m_new))
                    s_cur = jnp.exp(lvec - m_new)
                    d_n = d * s_old + s_cur
                    accs_n = [a * s_old + r * s_cur
                              for a, r in zip(accs, row)]
                    cur_n = jnp.where(active, d_e, cur_n)
                    m = jnp.where(active, m_new, m)
                    d = jnp.where(active, d_n, d)
                    accs = tuple(jnp.where(active, an, a)
                                 for an, a in zip(accs_n, accs))
                carry = (cur_n, m, d, accs)
            return carry

        carry = lax.fori_loop(0, (nb + 1) // 2, pair_body, carry)
        _wait(*bufs[0])
        return carry

    init = (jnp.int32(-1),
            jnp.full((16,), -jnp.inf, jnp.float32),
            jnp.zeros((16,), jnp.float32),
            tuple(jnp.zeros((16,), jnp.float32) for _ in range(8)))
    cur_n, m, d, accs = lax.fori_loop(0, nwnd, window_body, init)
    inv = 1.0 / d
    cl = jnp.clip(cur_n - base, 0, NPT - 1)

    @pl.when(cur_n >= 0)
    def _():
        for j in range(8):
            out_v[pl.ds(cl * D + 16 * j, 16)] = accs[j] * inv

    pltpu.sync_copy(out_v, out_hbm.at[pl.ds(base * D, NPT * D)])


@jax.jit
def _edge_attn(xl, xr_pad, src_s, dst_s, rowptr, att):
    mesh = plsc.VectorSubcoreMesh(core_axis_name="c", subcore_axis_name="s",
                                  num_cores=2, num_subcores=16)
    f = pl.kernel(
        _edge_body,
        out_type=jax.ShapeDtypeStruct((NPAD * D,), jnp.float32),
        mesh=mesh,
        scratch_types=[
            pltpu.VMEM((D,), jnp.float32),
            pltpu.VMEM((NPT * D,), jnp.float32),
            pltpu.VMEM((W,), jnp.int32),
            pltpu.VMEM((W,), jnp.int32),
            pltpu.VMEM((GC, D), jnp.float32),
            pltpu.VMEM((GC, D), jnp.float32),
            pltpu.VMEM((16,), jnp.int32),
            pltpu.VMEM((NPT * D,), jnp.float32),
            pltpu.SemaphoreType.DMA,
            pltpu.SemaphoreType.DMA,
        ],
    )
    return f(xl, xr_pad, src_s, dst_s, rowptr, att)


def kernel(x, edge_index, params):
    loop = jnp.arange(N, dtype=edge_index.dtype)
    src = jnp.concatenate([edge_index[0], loop])
    dst = jnp.concatenate([edge_index[1], loop])
    dst_s, src_s = lax.sort((dst, src), num_keys=1)
    rowptr = jnp.searchsorted(dst_s, jnp.minimum(jnp.arange(NPAD + 16), N),
                              side="left").astype(jnp.int32)
    src_s = jnp.concatenate([src_s, jnp.zeros((W,), jnp.int32)])
    dst_s = jnp.concatenate([dst_s, jnp.zeros((W,), jnp.int32)])

    h = x
    for (Wl, bl, Wr, br, att, bias, g, b) in params:
        xl, xr = _mm2(h, Wl, bl, Wr, br)
        xr_pad = jnp.pad(xr, ((0, NPAD - N), (0, 0))).reshape(NPAD * D)
        attn = _edge_attn(xl, xr_pad, src_s, dst_s, rowptr, att.reshape(D))
        h = _post(attn.reshape(NPAD, D)[:N], bias, g, b)
    return h


# speedup vs baseline: 1.0454x; 1.0454x over previous
import functools

import jax
import jax.numpy as jnp
from jax import lax
from jax.experimental import pallas as pl
from jax.experimental.pallas import tpu as pltpu
from jax.experimental.pallas import tpu_sc as plsc

N = 10000
E = 320000
D = 128
NEG_SLOPE = 0.2

NW = 32
NPT = 320
NPAD = NW * NPT
EDGES = E + N
W = 2048
GC = 16
NBLK = W // GC

_ROWS = 1000


def _mm2_body(x_ref, wl_ref, bl_ref, wr_ref, br_ref, xl_ref, xr_ref):
    x = x_ref[...]
    xl_ref[...] = jnp.dot(x, wl_ref[...], preferred_element_type=jnp.float32) + bl_ref[...]
    xr_ref[...] = jnp.dot(x, wr_ref[...], preferred_element_type=jnp.float32) + br_ref[...]


@jax.jit
def _mm2(x, Wl, bl, Wr, br):
    grid = (N // _ROWS,)
    return pl.pallas_call(
        _mm2_body,
        grid=grid,
        in_specs=[
            pl.BlockSpec((_ROWS, D), lambda i: (i, 0)),
            pl.BlockSpec((D, D), lambda i: (0, 0)),
            pl.BlockSpec((D,), lambda i: (0,)),
            pl.BlockSpec((D, D), lambda i: (0, 0)),
            pl.BlockSpec((D,), lambda i: (0,)),
        ],
        out_specs=[
            pl.BlockSpec((_ROWS, D), lambda i: (i, 0)),
            pl.BlockSpec((_ROWS, D), lambda i: (i, 0)),
        ],
        out_shape=[
            jax.ShapeDtypeStruct((N, D), jnp.float32),
            jax.ShapeDtypeStruct((N, D), jnp.float32),
        ],
    )(x, Wl, bl, Wr, br)


def _post_body(attn_ref, bias_ref, g_ref, b_ref, out_ref):
    o = attn_ref[...] + bias_ref[...]
    mu = jnp.mean(o, axis=-1, keepdims=True)
    var = jnp.mean((o - mu) ** 2, axis=-1, keepdims=True)
    o = (o - mu) / jnp.sqrt(var + 1e-5) * g_ref[...] + b_ref[...]
    out_ref[...] = jnp.maximum(o, 0.0)


@jax.jit
def _post(attn, bias, g, b):
    grid = (N // _ROWS,)
    return pl.pallas_call(
        _post_body,
        grid=grid,
        in_specs=[
            pl.BlockSpec((_ROWS, D), lambda i: (i, 0)),
            pl.BlockSpec((D,), lambda i: (0,)),
            pl.BlockSpec((D,), lambda i: (0,)),
            pl.BlockSpec((D,), lambda i: (0,)),
        ],
        out_specs=pl.BlockSpec((_ROWS, D), lambda i: (i, 0)),
        out_shape=jax.ShapeDtypeStruct((N, D), jnp.float32),
    )(attn, bias, g, b)


def _lanesum(v):
    lanes = lax.iota(jnp.int32, 16)
    for stride in (1, 2, 4, 8):
        v = v + jnp.take_along_axis(v, lanes ^ stride, axis=0)
    return v


def _edge_body(xl_hbm, xr_hbm, src_hbm, dst_hbm, rp_hbm, att_hbm, out_hbm,
               att_v, xr_v, srcw_v, dstw_v, rows0_v, rows1_v, rp_v, out_v,
               sem0, sem1):
    c = lax.axis_index("c")
    s = lax.axis_index("s")
    wid = s * 2 + c
    base = wid * NPT

    pltpu.sync_copy(att_hbm, att_v)
    pltpu.sync_copy(xr_hbm.at[pl.ds(base * D, NPT * D)], xr_v)
    pltpu.sync_copy(rp_hbm.at[pl.ds(base, 8)], rp_v.at[pl.ds(0, 8)])
    pltpu.sync_copy(rp_hbm.at[pl.ds(base + NPT, 8)], rp_v.at[pl.ds(8, 8)])
    rpv = rp_v[...]
    e0 = rpv[0]
    e1 = rpv[8]
    e0a = (e0 // 8) * 8
    att_regs = [att_v[pl.ds(16 * j, 16)] for j in range(8)]

    nwnd = (e1 - e0a + (W - 1)) // W

    bufs = ((rows0_v, sem0), (rows1_v, sem1))

    def _issue(b, buf, sem):
        start = jnp.minimum(b * GC, W - GC)
        pltpu.async_copy(xl_hbm.at[srcw_v[pl.ds(start, GC)]], buf, sem)

    def _wait(buf, sem):
        pltpu.make_async_copy(xl_hbm.at[pl.ds(0, GC), :], buf, sem).wait()

    def window_body(w, carry):
        wstart = e0a + w * W
        pltpu.sync_copy(src_hbm.at[pl.ds(wstart, W)], srcw_v)
        pltpu.sync_copy(dst_hbm.at[pl.ds(wstart, W)], dstw_v)
        nb = jnp.minimum(NBLK, (e1 - wstart + (GC - 1)) // GC)
        _issue(0, *bufs[0])

        def pair_body(p, carry):
            for h in (0, 1):
                b = 2 * p + h
                rows_v, sem = bufs[h]
                nxt_v, nxt_sem = bufs[1 - h]
                _issue(b + 1, nxt_v, nxt_sem)
                _wait(rows_v, sem)
                dvec = dstw_v[pl.ds(jnp.minimum(b * GC, W - GC), GC)]
                eid0 = wstart + b * GC
                cur_n, K, d, accs = carry
                for l in range(GC):
                    e_id = eid0 + l
                    active = jnp.logical_and(e_id >= e0, e_id < e1)
                    d_e = dvec[l]
                    nl = jnp.clip(d_e - base, 0, NPT - 1)
                    row = [rows_v[l, pl.ds(16 * j, 16)] for j in range(8)]
                    part = None
                    for j in range(8):
                        z = row[j] + xr_v[pl.ds(nl * D + 16 * j, 16)]
                        lz = jnp.maximum(z, NEG_SLOPE * z)
                        t = lz * att_regs[j]
                        part = t if part is None else part + t
                    lvec = _lanesum(part)
                    lvec = jnp.where(active, lvec,
                                     jnp.full((16,), -1e30, jnp.float32))
                    is_new = jnp.logical_and(active, d_e != cur_n)
                    do_fin = jnp.logical_and(is_new, cur_n >= 0)
                    cl = jnp.clip(cur_n - base, 0, NPT - 1)

                    @pl.when(do_fin)
                    def _():
                        inv = 1.0 / d
                        for j in range(8):
                            out_v[pl.ds(cl * D + 16 * j, 16)] = accs[j] * inv

                    K = jnp.where(is_new, lvec, K)
                    ex = jnp.exp(jnp.minimum(lvec - K, 60.0))
                    zm = jnp.where(is_new, jnp.zeros((16,), jnp.float32),
                                   jnp.ones((16,), jnp.float32))
                    d = d * zm + ex
                    accs = tuple(a * zm + r * ex
                                 for a, r in zip(accs, row))
                    cur_n = jnp.where(active, d_e, cur_n)
                carry = (cur_n, K, d, accs)
            return carry

        carry = lax.fori_loop(0, (nb + 1) // 2, pair_body, carry)
        _wait(*bufs[0])
        return carry

    init = (jnp.int32(-1),
            jnp.zeros((16,), jnp.float32),
            jnp.zeros((16,), jnp.float32),
            tuple(jnp.zeros((16,), jnp.float32) for _ in range(8)))
    cur_n, _K, d, accs = lax.fori_loop(0, nwnd, window_body, init)
    inv = 1.0 / d
    cl = jnp.clip(cur_n - base, 0, NPT - 1)

    @pl.when(cur_n >= 0)
    def _():
        for j in range(8):
            out_v[pl.ds(cl * D + 16 * j, 16)] = accs[j] * inv

    pltpu.sync_copy(out_v, out_hbm.at[pl.ds(base * D, NPT * D)])


@jax.jit
def _edge_attn(xl, xr_pad, src_s, dst_s, rowptr, att):
    mesh = plsc.VectorSubcoreMesh(core_axis_name="c", subcore_axis_name="s",
                                  num_cores=2, num_subcores=16)
    f = pl.kernel(
        _edge_body,
        out_type=jax.ShapeDtypeStruct((NPAD * D,), jnp.float32),
        mesh=mesh,
        scratch_types=[
            pltpu.VMEM((D,), jnp.float32),
            pltpu.VMEM((NPT * D,), jnp.float32),
            pltpu.VMEM((W,), jnp.int32),
            pltpu.VMEM((W,), jnp.int32),
            pltpu.VMEM((GC, D), jnp.float32),
            pltpu.VMEM((GC, D), jnp.float32),
            pltpu.VMEM((16,), jnp.int32),
            pltpu.VMEM((NPT * D,), jnp.float32),
            pltpu.SemaphoreType.DMA,
            pltpu.SemaphoreType.DMA,
        ],
    )
    return f(xl, xr_pad, src_s, dst_s, rowptr, att)


def kernel(x, edge_index, params):
    loop = jnp.arange(N, dtype=edge_index.dtype)
    src = jnp.concatenate([edge_index[0], loop])
    dst = jnp.concatenate([edge_index[1], loop])
    dst_s, src_s = lax.sort((dst, src), num_keys=1)
    rowptr = jnp.searchsorted(dst_s, jnp.minimum(jnp.arange(NPAD + 16), N),
                              side="left").astype(jnp.int32)
    src_s = jnp.concatenate([src_s, jnp.zeros((W,), jnp.int32)])
    dst_s = jnp.concatenate([dst_s, jnp.zeros((W,), jnp.int32)])

    h = x
    for (Wl, bl, Wr, br, att, bias, g, b) in params:
        xl, xr = _mm2(h, Wl, bl, Wr, br)
        xr_pad = jnp.pad(xr, ((0, NPAD - N), (0, 0))).reshape(NPAD * D)
        attn = _edge_attn(xl, xr_pad, src_s, dst_s, rowptr, att.reshape(D))
        h = _post(attn.reshape(NPAD, D)[:N], bias, g, b)
    return h


# speedup vs baseline: 1.4641x; 1.4005x over previous
import functools

import jax
import jax.numpy as jnp
from jax import lax
from jax.experimental import pallas as pl
from jax.experimental.pallas import tpu as pltpu
from jax.experimental.pallas import tpu_sc as plsc

N = 10000
E = 320000
D = 128
NEG_SLOPE = 0.2

NW = 32
NPT = 320
NPAD = NW * NPT
EDGES = E + N
W = 2048
GC = 16
NBLK = W // GC

_ROWS = 1000


def _mm2_body(x_ref, wl_ref, bl_ref, wr_ref, br_ref, xl_ref, xr_ref):
    x = x_ref[...]
    xl_ref[...] = jnp.dot(x, wl_ref[...], preferred_element_type=jnp.float32) + bl_ref[...]
    xr_ref[...] = jnp.dot(x, wr_ref[...], preferred_element_type=jnp.float32) + br_ref[...]


@jax.jit
def _mm2(x, Wl, bl, Wr, br):
    grid = (N // _ROWS,)
    return pl.pallas_call(
        _mm2_body,
        grid=grid,
        in_specs=[
            pl.BlockSpec((_ROWS, D), lambda i: (i, 0)),
            pl.BlockSpec((D, D), lambda i: (0, 0)),
            pl.BlockSpec((D,), lambda i: (0,)),
            pl.BlockSpec((D, D), lambda i: (0, 0)),
            pl.BlockSpec((D,), lambda i: (0,)),
        ],
        out_specs=[
            pl.BlockSpec((_ROWS, D), lambda i: (i, 0)),
            pl.BlockSpec((_ROWS, D), lambda i: (i, 0)),
        ],
        out_shape=[
            jax.ShapeDtypeStruct((N, D), jnp.float32),
            jax.ShapeDtypeStruct((N, D), jnp.float32),
        ],
    )(x, Wl, bl, Wr, br)


def _post_body(attn_ref, bias_ref, g_ref, b_ref, out_ref):
    o = attn_ref[...] + bias_ref[...]
    mu = jnp.mean(o, axis=-1, keepdims=True)
    var = jnp.mean((o - mu) ** 2, axis=-1, keepdims=True)
    o = (o - mu) / jnp.sqrt(var + 1e-5) * g_ref[...] + b_ref[...]
    out_ref[...] = jnp.maximum(o, 0.0)


@jax.jit
def _post(attn, bias, g, b):
    grid = (N // _ROWS,)
    return pl.pallas_call(
        _post_body,
        grid=grid,
        in_specs=[
            pl.BlockSpec((_ROWS, D), lambda i: (i, 0)),
            pl.BlockSpec((D,), lambda i: (0,)),
            pl.BlockSpec((D,), lambda i: (0,)),
            pl.BlockSpec((D,), lambda i: (0,)),
        ],
        out_specs=pl.BlockSpec((_ROWS, D), lambda i: (i, 0)),
        out_shape=jax.ShapeDtypeStruct((N, D), jnp.float32),
    )(attn, bias, g, b)


def _lanesum(v):
    lanes = lax.iota(jnp.int32, 16)
    for stride in (1, 2, 4, 8):
        v = v + jnp.take_along_axis(v, lanes ^ stride, axis=0)
    return v


def _edge_body(xl_hbm, xr_hbm, key_hbm, rp_hbm, att_hbm, out_hbm,
               att_v, xr_v, keyw_v, rows0_v, rows1_v, rp_v, out_v,
               sem0, sem1):
    c = lax.axis_index("c")
    s = lax.axis_index("s")
    wid = s * 2 + c
    base = wid * NPT

    pltpu.sync_copy(att_hbm, att_v)
    pltpu.sync_copy(xr_hbm.at[pl.ds(base * D, NPT * D)], xr_v)
    pltpu.sync_copy(rp_hbm.at[pl.ds(base, 8)], rp_v.at[pl.ds(0, 8)])
    pltpu.sync_copy(rp_hbm.at[pl.ds(base + NPT, 8)], rp_v.at[pl.ds(8, 8)])
    rpv = rp_v[...]
    e0 = rpv[0]
    e1 = rpv[8]
    e0a = (e0 // 8) * 8
    att_regs = [att_v[pl.ds(16 * j, 16)] for j in range(8)]

    nwnd = (e1 - e0a + (W - 1)) // W

    bufs = ((rows0_v, sem0), (rows1_v, sem1))

    def _issue(b, buf, sem):
        start = jnp.minimum(b * GC, W - GC)
        idx = jnp.bitwise_and(keyw_v[pl.ds(start, GC)], 16383)
        pltpu.async_copy(xl_hbm.at[idx], buf, sem)

    def _wait(buf, sem):
        pltpu.make_async_copy(xl_hbm.at[pl.ds(0, GC), :], buf, sem).wait()

    def window_body(w, carry):
        wstart = e0a + w * W
        pltpu.sync_copy(key_hbm.at[pl.ds(wstart, W)], keyw_v)
        nb = jnp.minimum(NBLK, (e1 - wstart + (GC - 1)) // GC)
        _issue(0, *bufs[0])

        def pair_body(p, carry):
            for h in (0, 1):
                b = 2 * p + h
                rows_v, sem = bufs[h]
                nxt_v, nxt_sem = bufs[1 - h]
                _issue(b + 1, nxt_v, nxt_sem)
                _wait(rows_v, sem)
                dvec = jnp.right_shift(
                    keyw_v[pl.ds(jnp.minimum(b * GC, W - GC), GC)], 14)
                eid0 = wstart + b * GC
                cur_n, K, d, accs = carry
                for l in range(GC):
                    e_id = eid0 + l
                    active = jnp.logical_and(e_id >= e0, e_id < e1)
                    d_e = dvec[l]
                    nl = jnp.clip(d_e - base, 0, NPT - 1)
                    row = [rows_v[l, pl.ds(16 * j, 16)] for j in range(8)]
                    part = None
                    for j in range(8):
                        z = row[j] + xr_v[pl.ds(nl * D + 16 * j, 16)]
                        lz = jnp.maximum(z, NEG_SLOPE * z)
                        t = lz * att_regs[j]
                        part = t if part is None else part + t
                    lvec = _lanesum(part)
                    lvec = jnp.where(active, lvec,
                                     jnp.full((16,), -1e30, jnp.float32))
                    is_new = jnp.logical_and(active, d_e != cur_n)
                    do_fin = jnp.logical_and(is_new, cur_n >= 0)
                    cl = jnp.clip(cur_n - base, 0, NPT - 1)

                    @pl.when(do_fin)
                    def _():
                        inv = 1.0 / d
                        for j in range(8):
                            out_v[pl.ds(cl * D + 16 * j, 16)] = accs[j] * inv

                    K = jnp.where(is_new, lvec, K)
                    ex = jnp.exp(jnp.minimum(lvec - K, 60.0))
                    zm = jnp.where(is_new, jnp.zeros((16,), jnp.float32),
                                   jnp.ones((16,), jnp.float32))
                    d = d * zm + ex
                    accs = tuple(a * zm + r * ex
                                 for a, r in zip(accs, row))
                    cur_n = jnp.where(active, d_e, cur_n)
                carry = (cur_n, K, d, accs)
            return carry

        carry = lax.fori_loop(0, (nb + 1) // 2, pair_body, carry)
        _wait(*bufs[0])
        return carry

    init = (jnp.int32(-1),
            jnp.zeros((16,), jnp.float32),
            jnp.zeros((16,), jnp.float32),
            tuple(jnp.zeros((16,), jnp.float32) for _ in range(8)))
    cur_n, _K, d, accs = lax.fori_loop(0, nwnd, window_body, init)
    inv = 1.0 / d
    cl = jnp.clip(cur_n - base, 0, NPT - 1)

    @pl.when(cur_n >= 0)
    def _():
        for j in range(8):
            out_v[pl.ds(cl * D + 16 * j, 16)] = accs[j] * inv

    pltpu.sync_copy(out_v, out_hbm.at[pl.ds(base * D, NPT * D)])


@jax.jit
def _edge_attn(xl, xr_pad, key_s, rowptr, att):
    mesh = plsc.VectorSubcoreMesh(core_axis_name="c", subcore_axis_name="s",
                                  num_cores=2, num_subcores=16)
    f = pl.kernel(
        _edge_body,
        out_type=jax.ShapeDtypeStruct((NPAD * D,), jnp.float32),
        mesh=mesh,
        scratch_types=[
            pltpu.VMEM((D,), jnp.float32),
            pltpu.VMEM((NPT * D,), jnp.float32),
            pltpu.VMEM((W,), jnp.int32),
            pltpu.VMEM((GC, D), jnp.float32),
            pltpu.VMEM((GC, D), jnp.float32),
            pltpu.VMEM((16,), jnp.int32),
            pltpu.VMEM((NPT * D,), jnp.float32),
            pltpu.SemaphoreType.DMA,
            pltpu.SemaphoreType.DMA,
        ],
    )
    return f(xl, xr_pad, key_s, rowptr, att)


def kernel(x, edge_index, params):
    loop = jnp.arange(N, dtype=edge_index.dtype)
    src = jnp.concatenate([edge_index[0], loop])
    dst = jnp.concatenate([edge_index[1], loop])
    key = jnp.concatenate([
        dst * 16384 + src,
        jnp.full((W,), (NPAD - 1) * 16384, jnp.int32),
    ])
    key_s = lax.sort(key)
    rowptr = jnp.searchsorted(
        key_s, jnp.minimum(jnp.arange(NPAD + 16), N) * 16384,
        side="left").astype(jnp.int32)

    h = x
    for (Wl, bl, Wr, br, att, bias, g, b) in params:
        xl, xr = _mm2(h, Wl, bl, Wr, br)
        xr_pad = jnp.pad(xr, ((0, NPAD - N), (0, 0))).reshape(NPAD * D)
        attn = _edge_attn(xl, xr_pad, key_s, rowptr, att.reshape(D))
        h = _post(attn.reshape(NPAD, D)[:N], bias, g, b)
    return h
